# Initial kernel scaffold; baseline (speedup 1.0000x reference)
import jax
import jax.numpy as jnp
from jax.experimental import pallas as pl


def kernel(input, means, sigmas, values, bias):
    raise NotImplementedError("write your pallas kernel here")


# speedup vs baseline: 105.1552x; 105.1552x over previous
import functools
import math

import jax
import jax.numpy as jnp
from jax import lax
from jax.experimental import pallas as pl
from jax.experimental.pallas import tpu as pltpu
from jax.experimental.pallas import tpu_sc as plsc

EPS = 1e-6
OUT_SIZE = 4096
IN_SIZE = 4096
B, K, C = 4, 8192, 4
RANK = 2
GADD, RADD = 2, 2
RR = 128.0

BK = B * K
L = (2 ** RANK + GADD + RADD) * C
N = BK * L

BLK = 2048
GRID = BK // BLK

NW = 32
NROWS = N // 128
ROWS_PER_TILE = NROWS // NW
CHUNK_ROWS = 64
NCHUNK = ROWS_PER_TILE // CHUNK_ROWS


@functools.cache
def _sample_consts():
    skey = jax.random.key(12345)
    s1, s2 = jax.random.split(skey)
    rng = jnp.array([float(OUT_SIZE), float(IN_SIZE)], dtype=jnp.float32)
    samp = jax.random.uniform(s1, (B, K, C, GADD, RANK), dtype=jnp.float32) * (1.0 - EPS)
    sampled = jnp.floor(samp * rng)
    rr = jax.random.uniform(s2, (B, K, C, RADD, RANK), dtype=jnp.float32) * (1.0 - EPS)
    rrs = rr * jnp.float32(RR)
    smp_rows = jnp.transpose(sampled.reshape(BK, C * GADD * RANK))
    rrs_rows = jnp.transpose(rrs.reshape(BK, C * RADD * RANK))
    return jax.device_put(smp_rows), jax.device_put(rrs_rows)


def _dense_body(m_ref, s_ref, v_ref, smp_ref, rrs_ref,
                val_ref, oix_ref, iix_ref, props_ref):
    j = pl.program_id(0)
    boff = (j * BLK // K) * IN_SIZE

    m = [m_ref[i, :] for i in range(2 * C)]
    inv = [1.0 / (EPS + s_ref[i, :]) for i in range(2 * C)]

    pts = []
    for c in range(C):
        m0, m1 = m[2 * c], m[2 * c + 1]
        f0, c0 = jnp.floor(m0), jnp.ceil(m0)
        f1, c1 = jnp.floor(m1), jnp.ceil(m1)
        pts += [(f0, f1), (f0, c1), (c0, f1), (c0, c1)]
        for g in range(GADD):
            r0 = (c * GADD + g) * RANK
            pts.append((smp_ref[r0, :], smp_ref[r0 + 1, :]))
        rnd0, rnd1 = jnp.round(m0), jnp.round(m1)
        half = jnp.float32(RR * 0.5)
        fs = jnp.float32(float(OUT_SIZE))
        lo0 = jnp.maximum(rnd0 - half, 0.0)
        lo0 = jnp.where(rnd0 + half > fs, fs - jnp.float32(RR), lo0)
        lo1 = jnp.maximum(rnd1 - half, 0.0)
        lo1 = jnp.where(rnd1 + half > fs, fs - jnp.float32(RR), lo1)
        for a in range(RADD):
            r0 = (c * RADD + a) * RANK
            pts.append((jnp.floor(rrs_ref[r0, :] + lo0),
                        jnp.floor(rrs_ref[r0 + 1, :] + lo1)))

    denom = [None] * C
    for l, (p0, p1) in enumerate(pts):
        for c2 in range(C):
            d0 = p0 - m[2 * c2]
            d1 = p1 - m[2 * c2 + 1]
            pr = jnp.exp(-0.5 * (d0 * d0 * inv[2 * c2] + d1 * d1 * inv[2 * c2 + 1]))
            props_ref[l * C + c2, :] = pr
            denom[c2] = pr if l == 0 else denom[c2] + pr
    w = [v_ref[c2, :] / denom[c2] for c2 in range(C)]

    for l, (p0, p1) in enumerate(pts):
        acc = props_ref[l * C + 0, :] * w[0]
        for c2 in range(1, C):
            acc = acc + props_ref[l * C + c2, :] * w[c2]
        val_ref[l, :] = acc
        oi = jnp.clip(p0.astype(jnp.int32), 0, OUT_SIZE - 1)
        ii = jnp.clip(p1.astype(jnp.int32), 0, IN_SIZE - 1)
        oix_ref[l, :] = oi + boff
        iix_ref[l, :] = ii + boff


def _dense_stage(means, sigmas, values):
    smp_rows, rrs_rows = _sample_consts()
    m_rows = jnp.transpose(means.reshape(BK, 2 * C))
    s_rows = jnp.transpose(sigmas.reshape(BK, 2 * C))
    v_rows = jnp.transpose(values.reshape(BK, C))
    spec = lambda q: pl.BlockSpec((q, BLK), lambda j: (0, j))
    val, oix, iix = pl.pallas_call(
        _dense_body,
        grid=(GRID,),
        in_specs=[spec(2 * C), spec(2 * C), spec(C),
                  spec(C * GADD * RANK), spec(C * RADD * RANK)],
        out_specs=[spec(L), spec(L), spec(L)],
        out_shape=[
            jax.ShapeDtypeStruct((L, BK), jnp.float32),
            jax.ShapeDtypeStruct((L, BK), jnp.int32),
            jax.ShapeDtypeStruct((L, BK), jnp.int32),
        ],
        scratch_shapes=[pltpu.VMEM((L * C, BLK), jnp.float32)],
    )(m_rows, s_rows, v_rows, smp_rows, rrs_rows)
    return val, oix, iix


def _sc_body(inp_hbm, oix_hbm, iix_hbm, val_hbm, zeros_hbm, part_hbm,
             inp_v, oix_v, iix_v, val_v, con_v, acc_sh):
    cid = lax.axis_index("c")
    sid = lax.axis_index("s")
    wid = cid * 16 + sid

    pltpu.sync_copy(inp_hbm, inp_v)

    @pl.when(sid == 0)
    def _init():
        pltpu.sync_copy(zeros_hbm, acc_sh)

    plsc.subcore_barrier()

    row0 = wid * ROWS_PER_TILE
    for ch in range(NCHUNK):
        r0 = row0 + ch * CHUNK_ROWS
        pltpu.sync_copy(oix_hbm.at[pl.ds(r0, CHUNK_ROWS)], oix_v)
        pltpu.sync_copy(iix_hbm.at[pl.ds(r0, CHUNK_ROWS)], iix_v)
        pltpu.sync_copy(val_hbm.at[pl.ds(r0, CHUNK_ROWS)], val_v)

        def body(r, carry):
            for q in range(8):
                cb = q * 16
                ii = iix_v[r, pl.ds(cb, 16)]
                g = plsc.load_gather(inp_v, [ii])
                con_v[r, pl.ds(cb, 16)] = g * val_v[r, pl.ds(cb, 16)]
            pltpu.sync_copy(con_v.at[r], acc_sh.at[oix_v.at[r]], add=True)
            return carry

        lax.fori_loop(0, CHUNK_ROWS, body, 0)

    plsc.subcore_barrier()

    @pl.when(sid == 0)
    def _flush():
        pltpu.sync_copy(acc_sh, part_hbm.at[cid])


def _sparse_stage(inp_flat, oix, iix, val):
    mesh = plsc.VectorSubcoreMesh(core_axis_name="c", subcore_axis_name="s")
    zeros = jnp.zeros((B * OUT_SIZE,), jnp.float32)
    run = pl.kernel(
        _sc_body,
        out_type=jax.ShapeDtypeStruct((2, B * OUT_SIZE), jnp.float32),
        mesh=mesh,
        compiler_params=pltpu.CompilerParams(needs_layout_passes=False),
        scratch_types=[
            pltpu.VMEM((B * IN_SIZE,), jnp.float32),
            pltpu.VMEM((CHUNK_ROWS, 128), jnp.int32),
            pltpu.VMEM((CHUNK_ROWS, 128), jnp.int32),
            pltpu.VMEM((CHUNK_ROWS, 128), jnp.float32),
            pltpu.VMEM((CHUNK_ROWS, 128), jnp.float32),
            pltpu.VMEM_SHARED((B * OUT_SIZE,), jnp.float32),
        ],
    )
    return run(inp_flat, oix, iix, val, zeros)


def kernel(input, means, sigmas, values, bias):
    val, oix, iix = _dense_stage(means, sigmas, values)
    part = _sparse_stage(
        input.reshape(B * IN_SIZE),
        oix.reshape(NROWS, 128),
        iix.reshape(NROWS, 128),
        val.reshape(NROWS, 128),
    )
    y = (part[0] + part[1]).reshape(B, OUT_SIZE)
    return y + bias[None, :]


# speedup vs baseline: 115.8616x; 1.1018x over previous
import functools

import jax
import jax.numpy as jnp
from jax import lax
from jax.experimental import pallas as pl
from jax.experimental.pallas import tpu as pltpu
from jax.experimental.pallas import tpu_sc as plsc

EPS = 1e-6
OUT_SIZE = 4096
IN_SIZE = 4096
B, K, C = 4, 8192, 4
RANK = 2
GADD, RADD = 2, 2
RR = 128.0

BK = B * K
L = (2 ** RANK + GADD + RADD) * C
N = BK * L

BLK = 2048
GRID = BK // BLK

NW = 32
NROWS = N // 128
ROWS_PER_TILE = NROWS // NW
CHUNK_ROWS = 64
NCHUNK = ROWS_PER_TILE // CHUNK_ROWS
ACC_ROWS = B * OUT_SIZE // 128


@functools.cache
def _sample_consts():
    skey = jax.random.key(12345)
    s1, s2 = jax.random.split(skey)
    rng = jnp.array([float(OUT_SIZE), float(IN_SIZE)], dtype=jnp.float32)
    samp = jax.random.uniform(s1, (B, K, C, GADD, RANK), dtype=jnp.float32) * (1.0 - EPS)
    sampled = jnp.floor(samp * rng)
    rr = jax.random.uniform(s2, (B, K, C, RADD, RANK), dtype=jnp.float32) * (1.0 - EPS)
    rrs = rr * jnp.float32(RR)
    smp_rows = jnp.transpose(sampled.reshape(BK, C * GADD * RANK))
    rrs_rows = jnp.transpose(rrs.reshape(BK, C * RADD * RANK))
    return jax.device_put(smp_rows), jax.device_put(rrs_rows)


def _dense_body(m_ref, s_ref, v_ref, smp_ref, rrs_ref,
                val_ref, oix_ref, iix_ref, props_ref):
    j = pl.program_id(0)
    boff = (j * BLK // K) * IN_SIZE

    m = [m_ref[i, :] for i in range(2 * C)]
    inv = [1.0 / (EPS + s_ref[i, :]) for i in range(2 * C)]

    pts = []
    for c in range(C):
        m0, m1 = m[2 * c], m[2 * c + 1]
        f0, c0 = jnp.floor(m0), jnp.ceil(m0)
        f1, c1 = jnp.floor(m1), jnp.ceil(m1)
        pts += [(f0, f1), (f0, c1), (c0, f1), (c0, c1)]
        for g in range(GADD):
            r0 = (c * GADD + g) * RANK
            pts.append((smp_ref[r0, :], smp_ref[r0 + 1, :]))
        rnd0, rnd1 = jnp.round(m0), jnp.round(m1)
        half = jnp.float32(RR * 0.5)
        fs = jnp.float32(float(OUT_SIZE))
        lo0 = jnp.maximum(rnd0 - half, 0.0)
        lo0 = jnp.where(rnd0 + half > fs, fs - jnp.float32(RR), lo0)
        lo1 = jnp.maximum(rnd1 - half, 0.0)
        lo1 = jnp.where(rnd1 + half > fs, fs - jnp.float32(RR), lo1)
        for a in range(RADD):
            r0 = (c * RADD + a) * RANK
            pts.append((jnp.floor(rrs_ref[r0, :] + lo0),
                        jnp.floor(rrs_ref[r0 + 1, :] + lo1)))

    denom = [None] * C
    for l, (p0, p1) in enumerate(pts):
        for c2 in range(C):
            d0 = p0 - m[2 * c2]
            d1 = p1 - m[2 * c2 + 1]
            pr = jnp.exp(-0.5 * (d0 * d0 * inv[2 * c2] + d1 * d1 * inv[2 * c2 + 1]))
            props_ref[l * C + c2, :] = pr
            denom[c2] = pr if l == 0 else denom[c2] + pr
    w = [v_ref[c2, :] / denom[c2] for c2 in range(C)]

    for l, (p0, p1) in enumerate(pts):
        acc = props_ref[l * C + 0, :] * w[0]
        for c2 in range(1, C):
            acc = acc + props_ref[l * C + c2, :] * w[c2]
        val_ref[l, :] = acc
        oi = jnp.clip(p0.astype(jnp.int32), 0, OUT_SIZE - 1)
        ii = jnp.clip(p1.astype(jnp.int32), 0, IN_SIZE - 1)
        oix_ref[l, :] = oi + boff
        iix_ref[l, :] = ii + boff


def _dense_stage(means, sigmas, values):
    smp_rows, rrs_rows = _sample_consts()
    m_rows = jnp.transpose(means.reshape(BK, 2 * C))
    s_rows = jnp.transpose(sigmas.reshape(BK, 2 * C))
    v_rows = jnp.transpose(values.reshape(BK, C))
    spec = lambda q: pl.BlockSpec((q, BLK), lambda j: (0, j))
    val, oix, iix = pl.pallas_call(
        _dense_body,
        grid=(GRID,),
        in_specs=[spec(2 * C), spec(2 * C), spec(C),
                  spec(C * GADD * RANK), spec(C * RADD * RANK)],
        out_specs=[spec(L), spec(L), spec(L)],
        out_shape=[
            jax.ShapeDtypeStruct((L, BK), jnp.float32),
            jax.ShapeDtypeStruct((L, BK), jnp.int32),
            jax.ShapeDtypeStruct((L, BK), jnp.int32),
        ],
        scratch_shapes=[pltpu.VMEM((L * C, BLK), jnp.float32)],
    )(m_rows, s_rows, v_rows, smp_rows, rrs_rows)
    return val, oix, iix


def _sc_body(inp_hbm, oix_hbm, iix_hbm, val_hbm, zeros_hbm, part_hbm,
             inp_v, acc_v, oix_v, iix_v, val_v, rowidx_v, acc_sh):
    cid = lax.axis_index("c")
    sid = lax.axis_index("s")
    wid = cid * 16 + sid

    pltpu.sync_copy(inp_hbm, inp_v)
    pltpu.sync_copy(zeros_hbm, acc_v)
    for q in range(8):
        rowidx_v[pl.ds(q * 16, 16)] = lax.iota(jnp.int32, 16) + q * 16

    @pl.when(sid == 0)
    def _init():
        pltpu.sync_copy(zeros_hbm, acc_sh)

    plsc.subcore_barrier()

    row0 = wid * ROWS_PER_TILE
    for ch in range(NCHUNK):
        r0 = row0 + ch * CHUNK_ROWS
        pltpu.sync_copy(oix_hbm.at[pl.ds(r0, CHUNK_ROWS)], oix_v)
        pltpu.sync_copy(iix_hbm.at[pl.ds(r0, CHUNK_ROWS)], iix_v)
        pltpu.sync_copy(val_hbm.at[pl.ds(r0, CHUNK_ROWS)], val_v)

        def body(r, carry):
            for q in range(8):
                cb = q * 16
                ii = iix_v[r, pl.ds(cb, 16)]
                g = plsc.load_gather(inp_v, [ii])
                con = g * val_v[r, pl.ds(cb, 16)]
                oi = oix_v[r, pl.ds(cb, 16)]
                plsc.addupdate_scatter(
                    acc_v, [lax.shift_right_logical(oi, 7),
                            lax.bitwise_and(oi, 127)], con)
            return carry

        lax.fori_loop(0, CHUNK_ROWS, body, 0)

    pltpu.sync_copy(acc_v, acc_sh.at[rowidx_v], add=True)

    plsc.subcore_barrier()

    @pl.when(sid == 0)
    def _flush():
        pltpu.sync_copy(acc_sh, part_hbm.at[cid])


def _sparse_stage(inp_flat, oix, iix, val):
    mesh = plsc.VectorSubcoreMesh(core_axis_name="c", subcore_axis_name="s")
    zeros = jnp.zeros((ACC_ROWS, 128), jnp.float32)
    run = pl.kernel(
        _sc_body,
        out_type=jax.ShapeDtypeStruct((2, ACC_ROWS, 128), jnp.float32),
        mesh=mesh,
        compiler_params=pltpu.CompilerParams(needs_layout_passes=False),
        scratch_types=[
            pltpu.VMEM((B * IN_SIZE,), jnp.float32),
            pltpu.VMEM((ACC_ROWS, 128), jnp.float32),
            pltpu.VMEM((CHUNK_ROWS, 128), jnp.int32),
            pltpu.VMEM((CHUNK_ROWS, 128), jnp.int32),
            pltpu.VMEM((CHUNK_ROWS, 128), jnp.float32),
            pltpu.VMEM((ACC_ROWS,), jnp.int32),
            pltpu.VMEM_SHARED((ACC_ROWS, 128), jnp.float32),
        ],
    )
    return run(inp_flat, oix, iix, val, zeros)


def kernel(input, means, sigmas, values, bias):
    val, oix, iix = _dense_stage(means, sigmas, values)
    part = _sparse_stage(
        input.reshape(B * IN_SIZE),
        oix.reshape(NROWS, 128),
        iix.reshape(NROWS, 128),
        val.reshape(NROWS, 128),
    )
    y = (part[0] + part[1]).reshape(B, OUT_SIZE)
    return y + bias[None, :]


# speedup vs baseline: 265.6325x; 2.2927x over previous
import functools

import jax
import jax.numpy as jnp
from jax import lax
from jax.experimental import pallas as pl
from jax.experimental.pallas import tpu as pltpu
from jax.experimental.pallas import tpu_sc as plsc

EPS = 1e-6
OUT_SIZE = 4096
IN_SIZE = 4096
B, K, C = 4, 8192, 4
RANK = 2
GADD, RADD = 2, 2
RR = 128.0

BK = B * K
L = (2 ** RANK + GADD + RADD) * C
N = BK * L

BLK = 2048
GRID = BK // BLK

NW = 32
NROWS = N // 128
ROWS_PER_TILE = NROWS // NW
CHUNK_ROWS = 64
NCHUNK = ROWS_PER_TILE // CHUNK_ROWS
ACC_ROWS = B * OUT_SIZE // 128


def _sample_consts_math():
    skey = jax.random.key(12345)
    s1, s2 = jax.random.split(skey)
    rng = jnp.array([float(OUT_SIZE), float(IN_SIZE)], dtype=jnp.float32)
    samp = jax.random.uniform(s1, (B, K, C, GADD, RANK), dtype=jnp.float32) * (1.0 - EPS)
    sampled = jnp.floor(samp * rng)
    rr = jax.random.uniform(s2, (B, K, C, RADD, RANK), dtype=jnp.float32) * (1.0 - EPS)
    rrs = rr * jnp.float32(RR)
    smp_rows = jnp.transpose(sampled.reshape(BK, C * GADD * RANK))
    rrs_rows = jnp.transpose(rrs.reshape(BK, C * RADD * RANK))
    return smp_rows, rrs_rows


def _precompute_sample_consts():
    import numpy as np
    try:
        try:
            dev = jax.local_devices(backend="cpu")[0]
        except RuntimeError:
            dev = None
        if dev is not None:
            with jax.default_device(dev):
                a, b = _sample_consts_math()
                return np.asarray(a), np.asarray(b)
        a, b = _sample_consts_math()
        return np.asarray(a), np.asarray(b)
    except Exception:
        return None


_SAMPLE_CONSTS = _precompute_sample_consts()


def _sample_consts():
    if _SAMPLE_CONSTS is not None:
        return jnp.asarray(_SAMPLE_CONSTS[0]), jnp.asarray(_SAMPLE_CONSTS[1])
    return _sample_consts_math()


def _dense_body(m_ref, s_ref, v_ref, smp_ref, rrs_ref,
                val_ref, pk_ref, props_ref):
    j = pl.program_id(0)
    boff = (j * BLK // K) * IN_SIZE

    m = [m_ref[i, :] for i in range(2 * C)]
    inv = [1.0 / (EPS + s_ref[i, :]) for i in range(2 * C)]

    pts = []
    for c in range(C):
        m0, m1 = m[2 * c], m[2 * c + 1]
        f0, c0 = jnp.floor(m0), jnp.ceil(m0)
        f1, c1 = jnp.floor(m1), jnp.ceil(m1)
        pts += [(f0, f1), (f0, c1), (c0, f1), (c0, c1)]
        for g in range(GADD):
            r0 = (c * GADD + g) * RANK
            pts.append((smp_ref[r0, :], smp_ref[r0 + 1, :]))
        rnd0, rnd1 = jnp.round(m0), jnp.round(m1)
        half = jnp.float32(RR * 0.5)
        fs = jnp.float32(float(OUT_SIZE))
        lo0 = jnp.maximum(rnd0 - half, 0.0)
        lo0 = jnp.where(rnd0 + half > fs, fs - jnp.float32(RR), lo0)
        lo1 = jnp.maximum(rnd1 - half, 0.0)
        lo1 = jnp.where(rnd1 + half > fs, fs - jnp.float32(RR), lo1)
        for a in range(RADD):
            r0 = (c * RADD + a) * RANK
            pts.append((jnp.floor(rrs_ref[r0, :] + lo0),
                        jnp.floor(rrs_ref[r0 + 1, :] + lo1)))

    denom = [None] * C
    for l, (p0, p1) in enumerate(pts):
        for c2 in range(C):
            d0 = p0 - m[2 * c2]
            d1 = p1 - m[2 * c2 + 1]
            pr = jnp.exp(-0.5 * (d0 * d0 * inv[2 * c2] + d1 * d1 * inv[2 * c2 + 1]))
            props_ref[l * C + c2, :] = pr
            denom[c2] = pr if l == 0 else denom[c2] + pr
    w = [v_ref[c2, :] / denom[c2] for c2 in range(C)]

    for l, (p0, p1) in enumerate(pts):
        acc = props_ref[l * C + 0, :] * w[0]
        for c2 in range(1, C):
            acc = acc + props_ref[l * C + c2, :] * w[c2]
        val_ref[l, :] = acc
        oi = jnp.clip(p0.astype(jnp.int32), 0, OUT_SIZE - 1)
        ii = jnp.clip(p1.astype(jnp.int32), 0, IN_SIZE - 1)
        pk_ref[l, :] = lax.bitwise_or(
            lax.shift_left(oi + boff, 14), ii + boff)


def _dense_stage(means, sigmas, values):
    smp_rows, rrs_rows = _sample_consts()
    m_rows = jnp.transpose(means.reshape(BK, 2 * C))
    s_rows = jnp.transpose(sigmas.reshape(BK, 2 * C))
    v_rows = jnp.transpose(values.reshape(BK, C))
    spec = lambda q: pl.BlockSpec((q, BLK), lambda j: (0, j))
    val, pk = pl.pallas_call(
        _dense_body,
        grid=(GRID,),
        in_specs=[spec(2 * C), spec(2 * C), spec(C),
                  spec(C * GADD * RANK), spec(C * RADD * RANK)],
        out_specs=[spec(L), spec(L)],
        out_shape=[
            jax.ShapeDtypeStruct((L, BK), jnp.float32),
            jax.ShapeDtypeStruct((L, BK), jnp.int32),
        ],
        scratch_shapes=[pltpu.VMEM((L * C, BLK), jnp.float32)],
    )(m_rows, s_rows, v_rows, smp_rows, rrs_rows)
    return val, pk


def _sc_body(inp_hbm, pk_hbm, val_hbm, zeros_hbm, part_hbm,
             inp_v, acc_v, pk_v, val_v, rowidx_v, acc_sh):
    cid = lax.axis_index("c")
    sid = lax.axis_index("s")
    wid = cid * 16 + sid

    pltpu.sync_copy(inp_hbm, inp_v)
    pltpu.sync_copy(zeros_hbm, acc_v)
    for q in range(8):
        rowidx_v[pl.ds(q * 16, 16)] = lax.iota(jnp.int32, 16) + q * 16

    @pl.when(sid == 0)
    def _init():
        pltpu.sync_copy(zeros_hbm, acc_sh)

    pltpu.sync_copy(pk_hbm.at[wid], pk_v)
    pltpu.sync_copy(val_hbm.at[wid], val_v)

    plsc.subcore_barrier()

    def body(i, carry):
        b0 = i * 64
        pks = [pk_v[pl.ds(b0 + g * 16, 16)] for g in range(4)]
        vvs = [val_v[pl.ds(b0 + g * 16, 16)] for g in range(4)]
        gs = [plsc.load_gather(inp_v, [lax.bitwise_and(pks[g], 16383)])
              for g in range(4)]
        for g in range(4):
            oi = lax.shift_right_logical(pks[g], 14)
            plsc.addupdate_scatter(
                acc_v, [lax.shift_right_logical(oi, 7),
                        lax.bitwise_and(oi, 127)], gs[g] * vvs[g])
        return carry

    lax.fori_loop(0, BK // 64, body, 0)

    pltpu.sync_copy(acc_v, acc_sh.at[rowidx_v], add=True)

    plsc.subcore_barrier()

    @pl.when(sid == 0)
    def _flush():
        pltpu.sync_copy(acc_sh, part_hbm.at[cid])


def _sparse_stage(inp_flat, pk, val):
    mesh = plsc.VectorSubcoreMesh(core_axis_name="c", subcore_axis_name="s")
    zeros = jnp.zeros((ACC_ROWS, 128), jnp.float32)
    run = pl.kernel(
        _sc_body,
        out_type=jax.ShapeDtypeStruct((2, ACC_ROWS, 128), jnp.float32),
        mesh=mesh,
        compiler_params=pltpu.CompilerParams(needs_layout_passes=False),
        scratch_types=[
            pltpu.VMEM((B * IN_SIZE,), jnp.float32),
            pltpu.VMEM((ACC_ROWS, 128), jnp.float32),
            pltpu.VMEM((BK,), jnp.int32),
            pltpu.VMEM((BK,), jnp.float32),
            pltpu.VMEM((ACC_ROWS,), jnp.int32),
            pltpu.VMEM_SHARED((ACC_ROWS, 128), jnp.float32),
        ],
    )
    return run(inp_flat, pk, val, zeros)


def kernel(input, means, sigmas, values, bias):
    val, pk = _dense_stage(means, sigmas, values)
    part = _sparse_stage(input.reshape(B * IN_SIZE), pk, val)
    y = (part[0] + part[1]).reshape(B, OUT_SIZE)
    return y + bias[None, :]


# speedup vs baseline: 300.5913x; 1.1316x over previous
import functools

import jax
import jax.numpy as jnp
from jax import lax
from jax.experimental import pallas as pl
from jax.experimental.pallas import tpu as pltpu
from jax.experimental.pallas import tpu_sc as plsc

EPS = 1e-6
OUT_SIZE = 4096
IN_SIZE = 4096
B, K, C = 4, 8192, 4
RANK = 2
GADD, RADD = 2, 2
RR = 128.0

BK = B * K
L = (2 ** RANK + GADD + RADD) * C
N = BK * L

BLK = 2048
GRID = BK // BLK

NW = 32
NROWS = N // 128
ROWS_PER_TILE = NROWS // NW
CHUNK_ROWS = 64
NCHUNK = ROWS_PER_TILE // CHUNK_ROWS
ACC_ROWS = B * OUT_SIZE // 128


def _sample_consts_math():
    skey = jax.random.key(12345)
    s1, s2 = jax.random.split(skey)
    rng = jnp.array([float(OUT_SIZE), float(IN_SIZE)], dtype=jnp.float32)
    samp = jax.random.uniform(s1, (B, K, C, GADD, RANK), dtype=jnp.float32) * (1.0 - EPS)
    sampled = jnp.floor(samp * rng)
    rr = jax.random.uniform(s2, (B, K, C, RADD, RANK), dtype=jnp.float32) * (1.0 - EPS)
    rrs = rr * jnp.float32(RR)
    smp_rows = jnp.transpose(sampled.reshape(BK, C * GADD * RANK))
    rrs_rows = jnp.transpose(rrs.reshape(BK, C * RADD * RANK))
    return smp_rows, rrs_rows


def _precompute_sample_consts():
    import numpy as np
    try:
        try:
            dev = jax.local_devices(backend="cpu")[0]
        except RuntimeError:
            dev = None
        if dev is not None:
            with jax.default_device(dev):
                a, b = _sample_consts_math()
                return np.asarray(a), np.asarray(b)
        a, b = _sample_consts_math()
        return np.asarray(a), np.asarray(b)
    except Exception:
        return None


_SAMPLE_CONSTS = _precompute_sample_consts()


def _sample_consts():
    if _SAMPLE_CONSTS is not None:
        return jnp.asarray(_SAMPLE_CONSTS[0]), jnp.asarray(_SAMPLE_CONSTS[1])
    return _sample_consts_math()


def _dense_body(m_ref, s_ref, v_ref, smp_ref, rrs_ref,
                val_ref, pk_ref, props_ref):
    m = [m_ref[i, :] for i in range(2 * C)]
    inv = [1.0 / (EPS + s_ref[i, :]) for i in range(2 * C)]

    pts = []
    for c in range(C):
        m0, m1 = m[2 * c], m[2 * c + 1]
        f0, c0 = jnp.floor(m0), jnp.ceil(m0)
        f1, c1 = jnp.floor(m1), jnp.ceil(m1)
        pts += [(f0, f1), (f0, c1), (c0, f1), (c0, c1)]
        for g in range(GADD):
            r0 = (c * GADD + g) * RANK
            pts.append((smp_ref[r0, :], smp_ref[r0 + 1, :]))
        rnd0, rnd1 = jnp.round(m0), jnp.round(m1)
        half = jnp.float32(RR * 0.5)
        fs = jnp.float32(float(OUT_SIZE))
        lo0 = jnp.maximum(rnd0 - half, 0.0)
        lo0 = jnp.where(rnd0 + half > fs, fs - jnp.float32(RR), lo0)
        lo1 = jnp.maximum(rnd1 - half, 0.0)
        lo1 = jnp.where(rnd1 + half > fs, fs - jnp.float32(RR), lo1)
        for a in range(RADD):
            r0 = (c * RADD + a) * RANK
            pts.append((jnp.floor(rrs_ref[r0, :] + lo0),
                        jnp.floor(rrs_ref[r0 + 1, :] + lo1)))

    denom = [None] * C
    for l, (p0, p1) in enumerate(pts):
        for c2 in range(C):
            d0 = p0 - m[2 * c2]
            d1 = p1 - m[2 * c2 + 1]
            pr = jnp.exp(-0.5 * (d0 * d0 * inv[2 * c2] + d1 * d1 * inv[2 * c2 + 1]))
            props_ref[l * C + c2, :] = pr
            denom[c2] = pr if l == 0 else denom[c2] + pr
    w = [v_ref[c2, :] / denom[c2] for c2 in range(C)]

    for l, (p0, p1) in enumerate(pts):
        acc = props_ref[l * C + 0, :] * w[0]
        for c2 in range(1, C):
            acc = acc + props_ref[l * C + c2, :] * w[c2]
        val_ref[l, :] = acc
        oi = jnp.clip(p0.astype(jnp.int32), 0, OUT_SIZE - 1)
        ii = jnp.clip(p1.astype(jnp.int32), 0, IN_SIZE - 1)
        pk_ref[l, :] = lax.bitwise_or(lax.shift_left(oi, 12), ii)


def _dense_stage(means, sigmas, values):
    smp_rows, rrs_rows = _sample_consts()
    m_rows = jnp.transpose(means.reshape(BK, 2 * C))
    s_rows = jnp.transpose(sigmas.reshape(BK, 2 * C))
    v_rows = jnp.transpose(values.reshape(BK, C))
    spec = lambda q: pl.BlockSpec((q, BLK), lambda j: (0, j))
    val, pk = pl.pallas_call(
        _dense_body,
        grid=(GRID,),
        in_specs=[spec(2 * C), spec(2 * C), spec(C),
                  spec(C * GADD * RANK), spec(C * RADD * RANK)],
        out_specs=[spec(L), spec(L)],
        out_shape=[
            jax.ShapeDtypeStruct((L, BK), jnp.float32),
            jax.ShapeDtypeStruct((L, BK), jnp.int32),
        ],
        scratch_shapes=[pltpu.VMEM((L * C, BLK), jnp.float32)],
    )(m_rows, s_rows, v_rows, smp_rows, rrs_rows)
    return val, pk


def _sc_body(inp_hbm, pk_hbm, val_hbm, zeros_hbm, part_hbm,
             inp_v, acc_v, pk_v, val_v, rowidx_v, acc_sh, sem):
    cid = lax.axis_index("c")
    sid = lax.axis_index("s")
    wid = cid * 16 + sid
    b = wid // 8
    l0 = (wid % 8) * 4

    cps = [pltpu.async_copy(inp_hbm.at[b], inp_v, sem),
           pltpu.async_copy(zeros_hbm, acc_v, sem)]
    for q in range(4):
        cps.append(pltpu.async_copy(
            pk_hbm.at[l0 + q, pl.ds(b * K, K)],
            pk_v.at[pl.ds(q * K, K)], sem))
        cps.append(pltpu.async_copy(
            val_hbm.at[l0 + q, pl.ds(b * K, K)],
            val_v.at[pl.ds(q * K, K)], sem))
    for q in range(2):
        rowidx_v[pl.ds(q * 16, 16)] = lax.iota(jnp.int32, 16) + (b * 32 + q * 16)

    for t in range(4):
        @pl.when(sid == t)
        def _init(_t=t):
            pltpu.sync_copy(zeros_hbm, acc_sh.at[pl.ds(_t * 32, 32)])

    for cp in cps:
        cp.wait()

    plsc.subcore_barrier()

    def body(i, carry):
        b0 = i * 64
        pks = [pk_v[pl.ds(b0 + g * 16, 16)] for g in range(4)]
        vvs = [val_v[pl.ds(b0 + g * 16, 16)] for g in range(4)]
        gs = [plsc.load_gather(inp_v, [lax.bitwise_and(pks[g], 4095)])
              for g in range(4)]
        for g in range(4):
            oi = lax.shift_right_logical(pks[g], 12)
            plsc.addupdate_scatter(
                acc_v, [lax.shift_right_logical(oi, 7),
                        lax.bitwise_and(oi, 127)], gs[g] * vvs[g])
        return carry

    lax.fori_loop(0, BK // 64, body, 0)

    pltpu.sync_copy(acc_v, acc_sh.at[rowidx_v], add=True)

    plsc.subcore_barrier()

    @pl.when(sid == 0)
    def _flush():
        pltpu.sync_copy(acc_sh, part_hbm.at[cid])


def _sparse_stage(inp2d, pk, val):
    mesh = plsc.VectorSubcoreMesh(core_axis_name="c", subcore_axis_name="s")
    zeros = jnp.zeros((32, 128), jnp.float32)
    run = pl.kernel(
        _sc_body,
        out_type=jax.ShapeDtypeStruct((2, ACC_ROWS, 128), jnp.float32),
        mesh=mesh,
        compiler_params=pltpu.CompilerParams(needs_layout_passes=False),
        scratch_types=[
            pltpu.VMEM((IN_SIZE,), jnp.float32),
            pltpu.VMEM((32, 128), jnp.float32),
            pltpu.VMEM((BK,), jnp.int32),
            pltpu.VMEM((BK,), jnp.float32),
            pltpu.VMEM((32,), jnp.int32),
            pltpu.VMEM_SHARED((ACC_ROWS, 128), jnp.float32),
            pltpu.SemaphoreType.DMA,
        ],
    )
    return run(inp2d, pk, val, zeros)


def kernel(input, means, sigmas, values, bias):
    val, pk = _dense_stage(means, sigmas, values)
    part = _sparse_stage(input, pk, val)
    y = (part[0] + part[1]).reshape(B, OUT_SIZE)
    return y + bias[None, :]


# speedup vs baseline: 303.4561x; 1.0095x over previous
import functools

import jax
import jax.numpy as jnp
from jax import lax
from jax.experimental import pallas as pl
from jax.experimental.pallas import tpu as pltpu
from jax.experimental.pallas import tpu_sc as plsc

EPS = 1e-6
OUT_SIZE = 4096
IN_SIZE = 4096
B, K, C = 4, 8192, 4
RANK = 2
GADD, RADD = 2, 2
RR = 128.0

BK = B * K
L = (2 ** RANK + GADD + RADD) * C
N = BK * L

BLK = 2048
GRID = BK // BLK
LOG2E = 1.4426950408889634

NW = 32
NROWS = N // 128
ROWS_PER_TILE = NROWS // NW
CHUNK_ROWS = 64
NCHUNK = ROWS_PER_TILE // CHUNK_ROWS
ACC_ROWS = B * OUT_SIZE // 128


def _sample_consts_math():
    skey = jax.random.key(12345)
    s1, s2 = jax.random.split(skey)
    rng = jnp.array([float(OUT_SIZE), float(IN_SIZE)], dtype=jnp.float32)
    samp = jax.random.uniform(s1, (B, K, C, GADD, RANK), dtype=jnp.float32) * (1.0 - EPS)
    sampled = jnp.floor(samp * rng)
    rr = jax.random.uniform(s2, (B, K, C, RADD, RANK), dtype=jnp.float32) * (1.0 - EPS)
    rrs = rr * jnp.float32(RR)
    smp_rows = jnp.transpose(sampled.reshape(BK, C * GADD * RANK))
    rrs_rows = jnp.transpose(rrs.reshape(BK, C * RADD * RANK))
    return smp_rows, rrs_rows


def _precompute_sample_consts():
    import numpy as np
    try:
        try:
            dev = jax.local_devices(backend="cpu")[0]
        except RuntimeError:
            dev = None
        if dev is not None:
            with jax.default_device(dev):
                a, b = _sample_consts_math()
                return np.asarray(a), np.asarray(b)
        a, b = _sample_consts_math()
        return np.asarray(a), np.asarray(b)
    except Exception:
        return None


_SAMPLE_CONSTS = _precompute_sample_consts()


def _sample_consts():
    if _SAMPLE_CONSTS is not None:
        return jnp.asarray(_SAMPLE_CONSTS[0]), jnp.asarray(_SAMPLE_CONSTS[1])
    return _sample_consts_math()


def _dense_body(m_ref, s_ref, v_ref, smp_ref, rrs_ref,
                val_ref, pk_ref, props_ref):
    m = [m_ref[i, :] for i in range(2 * C)]
    inv = [jnp.float32(-0.5 * LOG2E) / (EPS + s_ref[i, :]) for i in range(2 * C)]

    pts = []
    for c in range(C):
        m0, m1 = m[2 * c], m[2 * c + 1]
        f0, c0 = jnp.floor(m0), jnp.ceil(m0)
        f1, c1 = jnp.floor(m1), jnp.ceil(m1)
        pts += [(f0, f1), (f0, c1), (c0, f1), (c0, c1)]
        for g in range(GADD):
            r0 = (c * GADD + g) * RANK
            pts.append((smp_ref[r0, :], smp_ref[r0 + 1, :]))
        rnd0, rnd1 = jnp.round(m0), jnp.round(m1)
        half = jnp.float32(RR * 0.5)
        fs = jnp.float32(float(OUT_SIZE))
        lo0 = jnp.maximum(rnd0 - half, 0.0)
        lo0 = jnp.where(rnd0 + half > fs, fs - jnp.float32(RR), lo0)
        lo1 = jnp.maximum(rnd1 - half, 0.0)
        lo1 = jnp.where(rnd1 + half > fs, fs - jnp.float32(RR), lo1)
        for a in range(RADD):
            r0 = (c * RADD + a) * RANK
            pts.append((jnp.floor(rrs_ref[r0, :] + lo0),
                        jnp.floor(rrs_ref[r0 + 1, :] + lo1)))

    denom = [None] * C
    for l, (p0, p1) in enumerate(pts):
        for c2 in range(C):
            d0 = p0 - m[2 * c2]
            d1 = p1 - m[2 * c2 + 1]
            pr = jnp.exp2(d0 * d0 * inv[2 * c2] + d1 * d1 * inv[2 * c2 + 1])
            props_ref[l * C + c2, :] = pr
            denom[c2] = pr if l == 0 else denom[c2] + pr
    w = [v_ref[c2, :] / denom[c2] for c2 in range(C)]

    for l, (p0, p1) in enumerate(pts):
        acc = props_ref[l * C + 0, :] * w[0]
        for c2 in range(1, C):
            acc = acc + props_ref[l * C + c2, :] * w[c2]
        val_ref[l, :] = acc
        oi = jnp.clip(p0.astype(jnp.int32), 0, OUT_SIZE - 1)
        ii = jnp.clip(p1.astype(jnp.int32), 0, IN_SIZE - 1)
        pk_ref[l, :] = lax.bitwise_or(lax.shift_left(oi, 12), ii)


def _dense_stage(means, sigmas, values):
    smp_rows, rrs_rows = _sample_consts()
    m_rows = jnp.transpose(means.reshape(BK, 2 * C))
    s_rows = jnp.transpose(sigmas.reshape(BK, 2 * C))
    v_rows = jnp.transpose(values.reshape(BK, C))
    spec = lambda q: pl.BlockSpec((q, BLK), lambda j: (0, j))
    val, pk = pl.pallas_call(
        _dense_body,
        grid=(GRID,),
        in_specs=[spec(2 * C), spec(2 * C), spec(C),
                  spec(C * GADD * RANK), spec(C * RADD * RANK)],
        out_specs=[spec(L), spec(L)],
        out_shape=[
            jax.ShapeDtypeStruct((L, BK), jnp.float32),
            jax.ShapeDtypeStruct((L, BK), jnp.int32),
        ],
        scratch_shapes=[pltpu.VMEM((L * C, BLK), jnp.float32)],
    )(m_rows, s_rows, v_rows, smp_rows, rrs_rows)
    return val, pk


def _sc_body(inp_hbm, pk_hbm, val_hbm, zeros_hbm, part_hbm,
             inp_v, acc_v, pk_v, val_v, rowidx_v, acc_sh, sem):
    cid = lax.axis_index("c")
    sid = lax.axis_index("s")
    wid = cid * 16 + sid
    b = wid // 8
    l0 = (wid % 8) * 4

    cps = [pltpu.async_copy(inp_hbm.at[b], inp_v, sem),
           pltpu.async_copy(zeros_hbm, acc_v, sem)]
    for q in range(4):
        cps.append(pltpu.async_copy(
            pk_hbm.at[l0 + q, pl.ds(b * K, K)],
            pk_v.at[pl.ds(q * K, K)], sem))
        cps.append(pltpu.async_copy(
            val_hbm.at[l0 + q, pl.ds(b * K, K)],
            val_v.at[pl.ds(q * K, K)], sem))
    for q in range(2):
        rowidx_v[pl.ds(q * 16, 16)] = lax.iota(jnp.int32, 16) + (b * 32 + q * 16)

    for t in range(4):
        @pl.when(sid == t)
        def _init(_t=t):
            pltpu.sync_copy(zeros_hbm, acc_sh.at[pl.ds(_t * 32, 32)])

    for cp in cps:
        cp.wait()

    plsc.subcore_barrier()

    def body(i, carry):
        b0 = i * 64
        pks = [pk_v[pl.ds(b0 + g * 16, 16)] for g in range(4)]
        vvs = [val_v[pl.ds(b0 + g * 16, 16)] for g in range(4)]
        gs = [plsc.load_gather(inp_v, [lax.bitwise_and(pks[g], 4095)])
              for g in range(4)]
        for g in range(4):
            oi = lax.shift_right_logical(pks[g], 12)
            plsc.addupdate_scatter(
                acc_v, [lax.shift_right_logical(oi, 7),
                        lax.bitwise_and(oi, 127)], gs[g] * vvs[g])
        return carry

    lax.fori_loop(0, BK // 64, body, 0)

    pltpu.sync_copy(acc_v, acc_sh.at[rowidx_v], add=True)

    plsc.subcore_barrier()

    @pl.when(sid == 0)
    def _flush():
        pltpu.sync_copy(acc_sh, part_hbm.at[cid])


def _sparse_stage(inp2d, pk, val):
    mesh = plsc.VectorSubcoreMesh(core_axis_name="c", subcore_axis_name="s")
    zeros = jnp.zeros((32, 128), jnp.float32)
    run = pl.kernel(
        _sc_body,
        out_type=jax.ShapeDtypeStruct((2, ACC_ROWS, 128), jnp.float32),
        mesh=mesh,
        compiler_params=pltpu.CompilerParams(needs_layout_passes=False),
        scratch_types=[
            pltpu.VMEM((IN_SIZE,), jnp.float32),
            pltpu.VMEM((32, 128), jnp.float32),
            pltpu.VMEM((BK,), jnp.int32),
            pltpu.VMEM((BK,), jnp.float32),
            pltpu.VMEM((32,), jnp.int32),
            pltpu.VMEM_SHARED((ACC_ROWS, 128), jnp.float32),
            pltpu.SemaphoreType.DMA,
        ],
    )
    return run(inp2d, pk, val, zeros)


def kernel(input, means, sigmas, values, bias):
    val, pk = _dense_stage(means, sigmas, values)
    part = _sparse_stage(input, pk, val)
    y = (part[0] + part[1]).reshape(B, OUT_SIZE)
    return y + bias[None, :]
